# baseline (speedup 1.0000x reference)
import functools

import jax
import jax.numpy as jnp
from jax import lax
from jax.experimental import pallas as pl
from jax.experimental.pallas import tpu as pltpu
from jax.experimental.pallas import tpu_sc as plsc

N = 10000
E = 320000
D = 128
NTILES = 16
CHUNK = 64
DEPTH = 4
EPT = E // NTILES
EPT_PAD = 20480
NCHUNK = EPT_PAD // CHUNK
IBLK = 16
NBLK = NCHUNK // IBLK
ROWS = 640
ACCR = N + 8

_f32 = jnp.float32


def _sc_body(xt, si, di, zf, sums, *refs):
    idx_s, idx_d = refs[0], refs[1]
    idxbs = refs[2:2 + DEPTH]
    rows = refs[2 + DEPTH:2 + 2 * DEPTH]
    acc = refs[2 + 2 * DEPTH]
    sem_g = refs[3 + 2 * DEPTH:3 + 3 * DEPTH]
    sem_s = refs[3 + 3 * DEPTH:3 + 4 * DEPTH]
    c = lax.axis_index("c")
    s = lax.axis_index("s")
    base = jnp.minimum(s * ROWS, N - ROWS)

    pltpu.sync_copy(zf.at[pl.ds(base, ROWS)], acc.at[pl.ds(base, ROWS)])
    plsc.subcore_barrier()

    @pl.loop(0, NBLK)
    def _(blk):
        pltpu.sync_copy(si.at[c, s, pl.ds(blk * IBLK, IBLK)], idx_s)
        pltpu.sync_copy(di.at[c, s, pl.ds(blk * IBLK, IBLK)], idx_d)

        @pl.loop(0, IBLK // DEPTH)
        def _(p):
            for b in range(DEPTH):
                for k in range(CHUNK // 16):
                    idxbs[b][pl.ds(k * 16, 16)] = (
                        idx_d[DEPTH * p + b, pl.ds(k * 16, 16)])
            gs = [pltpu.async_copy(xt.at[idx_s.at[DEPTH * p + b]],
                                   rows[b], sem_g[b])
                  for b in range(DEPTH)]
            ss = []
            for b in range(DEPTH):
                gs[b].wait()
                ss.append(pltpu.async_copy(rows[b], acc.at[idxbs[b]],
                                           sem_s[b], add=True))
            for b in range(DEPTH):
                ss[b].wait()

    plsc.subcore_barrier()
    obase = c * N + base
    pltpu.sync_copy(acc.at[pl.ds(base, ROWS)], sums.at[pl.ds(obase, ROWS)])


def _make_sc():
    return pl.kernel(
        _sc_body,
        out_type=[jax.ShapeDtypeStruct((2 * N, D), _f32)],
        mesh=plsc.VectorSubcoreMesh(core_axis_name="c", subcore_axis_name="s",
                                    num_cores=2, num_subcores=NTILES),
        scratch_types=(
            [pltpu.VMEM((IBLK, CHUNK), jnp.int32),
             pltpu.VMEM((IBLK, CHUNK), jnp.int32)]
            + [pltpu.VMEM((CHUNK,), jnp.int32)] * DEPTH
            + [pltpu.VMEM((CHUNK, D), _f32)] * DEPTH
            + [pltpu.VMEM_SHARED((ACCR, D), _f32)]
            + [pltpu.SemaphoreType.DMA] * (2 * DEPTH)
        ),
    )


_make_sc = functools.lru_cache(_make_sc)


def _cnt_body(di, zf, ones_h, cnts, idx_d, idxb0, idxb1, ones_v, cacc,
              sem_s0, sem_s1):
    c = lax.axis_index("c")
    s = lax.axis_index("s")
    base = jnp.minimum(s * ROWS, N - ROWS)

    pltpu.sync_copy(zf.at[pl.ds(base, ROWS)], cacc.at[pl.ds(base, ROWS)])
    pltpu.sync_copy(ones_h, ones_v)
    plsc.subcore_barrier()

    @pl.loop(0, NBLK)
    def _(blk):
        pltpu.sync_copy(di.at[c, s, pl.ds(blk * IBLK, IBLK)], idx_d)

        @pl.loop(0, IBLK // 2)
        def _(p):
            for k in range(CHUNK // 16):
                idxb0[pl.ds(k * 16, 16)] = idx_d[2 * p, pl.ds(k * 16, 16)]
                idxb1[pl.ds(k * 16, 16)] = idx_d[2 * p + 1, pl.ds(k * 16, 16)]
            s0 = pltpu.async_copy(ones_v, cacc.at[idxb0], sem_s0, add=True)
            s1 = pltpu.async_copy(ones_v, cacc.at[idxb1], sem_s1, add=True)
            s0.wait()
            s1.wait()

    plsc.subcore_barrier()
    obase = c * N + base
    pltpu.sync_copy(cacc.at[pl.ds(base, ROWS)], cnts.at[pl.ds(obase, ROWS)])


def _make_cnt():
    return pl.kernel(
        _cnt_body,
        out_type=[jax.ShapeDtypeStruct((2 * N, D), _f32)],
        mesh=plsc.VectorSubcoreMesh(core_axis_name="c", subcore_axis_name="s",
                                    num_cores=2, num_subcores=NTILES),
        scratch_types=[
            pltpu.VMEM((IBLK, CHUNK), jnp.int32),
            pltpu.VMEM((CHUNK,), jnp.int32),
            pltpu.VMEM((CHUNK,), jnp.int32),
            pltpu.VMEM((CHUNK, D), _f32),
            pltpu.VMEM_SHARED((ACCR, D), _f32),
            pltpu.SemaphoreType.DMA,
            pltpu.SemaphoreType.DMA,
        ],
    )


_make_cnt = functools.lru_cache(_make_cnt)


def _tc_body(with_post, *refs):
    if with_post:
        (x, sm, cnt, Ws, bs, Wd, bd, Wu, bu, g, be, Wp, bp, out) = refs
    else:
        (x, sm, cnt, Ws, bs, Wd, bd, Wu, bu, g, be, out) = refs
    c = jnp.maximum(cnt[:, 0:1], 1.0)
    aggr = sm[...] / c
    hv = jnp.dot(x[...], Wd[...], preferred_element_type=_f32) + bd[...]
    hu = jnp.dot(aggr, Ws[...], preferred_element_type=_f32) + bs[...]
    y = jnp.dot(jnp.concatenate([hv, hu], axis=1), Wu[...],
                preferred_element_type=_f32) + bu[...]
    m = jnp.mean(y, axis=0, keepdims=True)
    v = jnp.mean((y - m) ** 2, axis=0, keepdims=True)
    y = (y - m) / jnp.sqrt(v + 1.0) * g[...] + be[...]
    y = jnp.where(y >= 0, y, 0.01 * y)
    if with_post:
        y = jnp.dot(y, Wp[...], preferred_element_type=_f32) + bp[...]
    out[...] = y


def _full(a):
    return pl.BlockSpec(a.shape, lambda i: (0,) * a.ndim)


def _half(half, w):
    return pl.BlockSpec((N, w), lambda i: (half, 0))


def _tc_stage(with_post, half, x, sm, cnt, *args):
    width = 16 if with_post else D
    in_specs = ([_full(x), _half(half, D), _half(half, D)]
                + [_full(a) for a in args])
    return pl.pallas_call(
        functools.partial(_tc_body, with_post),
        grid=(1,),
        out_shape=jax.ShapeDtypeStruct((N, width), _f32),
        in_specs=in_specs,
        out_specs=pl.BlockSpec((N, width), lambda i: (0, 0)),
    )(x, sm, cnt, *args)


def kernel(x_n0, x_n1, edge_index_a, edge_index_b, W1a_src, b1a_src, W1a_dst, b1a_dst, W1a_upd, b1a_upd, W1b_src, b1b_src, W1b_dst, b1b_dst, W1b_upd, b1b_upd, W2a_src, b2a_src, W2a_dst, b2a_dst, W2a_upd, b2a_upd, W2b_src, b2b_src, W2b_dst, b2b_dst, W2b_upd, b2b_upd, g_bn1_n0, be_bn1_n0, g_bn1_n1, be_bn1_n1, g_bn2_n0, be_bn2_n0, g_bn2_n1, be_bn2_n1, W_post_n0, b_post_n0, W_post_n1, b_post_n1):
    def padidx(v, fill):
        v = v.astype(jnp.int32).reshape(NTILES, EPT)
        return jnp.pad(v, ((0, 0), (0, EPT_PAD - EPT)), constant_values=fill)

    si = jnp.stack([padidx(edge_index_a[0], 0),
                    padidx(edge_index_b[0] + N, 0)]
                   ).reshape(2, NTILES, NCHUNK, CHUNK)
    di = jnp.stack([padidx(edge_index_a[1], N),
                    padidx(edge_index_b[1], N)]
                   ).reshape(2, NTILES, NCHUNK, CHUNK)
    zf = jnp.zeros((N, D), _f32)
    row = lambda v: v.reshape(1, -1)

    xt1 = jnp.concatenate([x_n0, x_n1], axis=0)
    (sums1,) = _make_sc()(xt1, si, di, zf)
    (cnts,) = _make_cnt()(di, zf, jnp.ones((CHUNK, D), _f32))

    h1 = _tc_stage(False, 0, x_n1, sums1, cnts,
                   W1a_src, row(b1a_src), W1a_dst, row(b1a_dst),
                   W1a_upd, row(b1a_upd), row(g_bn1_n1), row(be_bn1_n1))
    h0 = _tc_stage(False, 1, x_n0, sums1, cnts,
                   W1b_src, row(b1b_src), W1b_dst, row(b1b_dst),
                   W1b_upd, row(b1b_upd), row(g_bn1_n0), row(be_bn1_n0))

    xt2 = jnp.concatenate([h0, h1], axis=0)
    (sums2,) = _make_sc()(xt2, si, di, zf)

    out1 = _tc_stage(True, 0, h1, sums2, cnts,
                     W2a_src, row(b2a_src), W2a_dst, row(b2a_dst),
                     W2a_upd, row(b2a_upd), row(g_bn2_n1), row(be_bn2_n1),
                     W_post_n1, row(b_post_n1))
    out0 = _tc_stage(True, 1, h0, sums2, cnts,
                     W2b_src, row(b2b_src), W2b_dst, row(b2b_dst),
                     W2b_upd, row(b2b_upd), row(g_bn2_n0), row(be_bn2_n0),
                     W_post_n0, row(b_post_n0))
    return out0, out1


# speedup vs baseline: 1.0208x; 1.0208x over previous
import functools

import jax
import jax.numpy as jnp
from jax import lax
from jax.experimental import pallas as pl
from jax.experimental.pallas import tpu as pltpu
from jax.experimental.pallas import tpu_sc as plsc

N = 10000
E = 320000
D = 128
NTILES = 16
CHUNK = 128
DEPTH = 2
EPT = E // NTILES
EPT_PAD = 20480
NCHUNK = EPT_PAD // CHUNK
IBLK = 16
NBLK = NCHUNK // IBLK
ROWS = 640
ACCR = N + 8

_f32 = jnp.float32


def _sc_body(xt, si, di, zf, sums, *refs):
    idx_s, idx_d = refs[0], refs[1]
    idxbs = refs[2:2 + DEPTH]
    rows = refs[2 + DEPTH:2 + 2 * DEPTH]
    acc = refs[2 + 2 * DEPTH]
    sem_g = refs[3 + 2 * DEPTH:3 + 3 * DEPTH]
    sem_s = refs[3 + 3 * DEPTH:3 + 4 * DEPTH]
    c = lax.axis_index("c")
    s = lax.axis_index("s")
    base = jnp.minimum(s * ROWS, N - ROWS)

    pltpu.sync_copy(zf.at[pl.ds(base, ROWS)], acc.at[pl.ds(base, ROWS)])
    plsc.subcore_barrier()

    @pl.loop(0, NBLK)
    def _(blk):
        pltpu.sync_copy(si.at[c, s, pl.ds(blk * IBLK, IBLK)], idx_s)
        pltpu.sync_copy(di.at[c, s, pl.ds(blk * IBLK, IBLK)], idx_d)

        @pl.loop(0, IBLK // DEPTH)
        def _(p):
            for b in range(DEPTH):
                for k in range(CHUNK // 16):
                    idxbs[b][pl.ds(k * 16, 16)] = (
                        idx_d[DEPTH * p + b, pl.ds(k * 16, 16)])
            gs = [pltpu.async_copy(xt.at[idx_s.at[DEPTH * p + b]],
                                   rows[b], sem_g[b])
                  for b in range(DEPTH)]
            ss = []
            for b in range(DEPTH):
                gs[b].wait()
                ss.append(pltpu.async_copy(rows[b], acc.at[idxbs[b]],
                                           sem_s[b], add=True))
            for b in range(DEPTH):
                ss[b].wait()

    plsc.subcore_barrier()
    obase = c * N + base
    pltpu.sync_copy(acc.at[pl.ds(base, ROWS)], sums.at[pl.ds(obase, ROWS)])


def _make_sc():
    return pl.kernel(
        _sc_body,
        out_type=[jax.ShapeDtypeStruct((2 * N, D), _f32)],
        mesh=plsc.VectorSubcoreMesh(core_axis_name="c", subcore_axis_name="s",
                                    num_cores=2, num_subcores=NTILES),
        scratch_types=(
            [pltpu.VMEM((IBLK, CHUNK), jnp.int32),
             pltpu.VMEM((IBLK, CHUNK), jnp.int32)]
            + [pltpu.VMEM((CHUNK,), jnp.int32)] * DEPTH
            + [pltpu.VMEM((CHUNK, D), _f32)] * DEPTH
            + [pltpu.VMEM_SHARED((ACCR, D), _f32)]
            + [pltpu.SemaphoreType.DMA] * (2 * DEPTH)
        ),
    )


_make_sc = functools.lru_cache(_make_sc)


def _cnt_body(di, zf, ones_h, cnts, idx_d, idxb0, idxb1, ones_v, cacc,
              sem_s0, sem_s1):
    c = lax.axis_index("c")
    s = lax.axis_index("s")
    base = jnp.minimum(s * ROWS, N - ROWS)

    pltpu.sync_copy(zf.at[pl.ds(base, ROWS)], cacc.at[pl.ds(base, ROWS)])
    pltpu.sync_copy(ones_h, ones_v)
    plsc.subcore_barrier()

    @pl.loop(0, NBLK)
    def _(blk):
        pltpu.sync_copy(di.at[c, s, pl.ds(blk * IBLK, IBLK)], idx_d)

        @pl.loop(0, IBLK // 2)
        def _(p):
            for k in range(CHUNK // 16):
                idxb0[pl.ds(k * 16, 16)] = idx_d[2 * p, pl.ds(k * 16, 16)]
                idxb1[pl.ds(k * 16, 16)] = idx_d[2 * p + 1, pl.ds(k * 16, 16)]
            s0 = pltpu.async_copy(ones_v, cacc.at[idxb0], sem_s0, add=True)
            s1 = pltpu.async_copy(ones_v, cacc.at[idxb1], sem_s1, add=True)
            s0.wait()
            s1.wait()

    plsc.subcore_barrier()
    obase = c * N + base
    pltpu.sync_copy(cacc.at[pl.ds(base, ROWS)], cnts.at[pl.ds(obase, ROWS)])


def _make_cnt():
    return pl.kernel(
        _cnt_body,
        out_type=[jax.ShapeDtypeStruct((2 * N, D), _f32)],
        mesh=plsc.VectorSubcoreMesh(core_axis_name="c", subcore_axis_name="s",
                                    num_cores=2, num_subcores=NTILES),
        scratch_types=[
            pltpu.VMEM((IBLK, CHUNK), jnp.int32),
            pltpu.VMEM((CHUNK,), jnp.int32),
            pltpu.VMEM((CHUNK,), jnp.int32),
            pltpu.VMEM((CHUNK, D), _f32),
            pltpu.VMEM_SHARED((ACCR, D), _f32),
            pltpu.SemaphoreType.DMA,
            pltpu.SemaphoreType.DMA,
        ],
    )


_make_cnt = functools.lru_cache(_make_cnt)


def _tc_body(with_post, *refs):
    if with_post:
        (x, sm, cnt, Ws, bs, Wd, bd, Wu, bu, g, be, Wp, bp, out) = refs
    else:
        (x, sm, cnt, Ws, bs, Wd, bd, Wu, bu, g, be, out) = refs
    c = jnp.maximum(cnt[:, 0:1], 1.0)
    aggr = sm[...] / c
    hv = jnp.dot(x[...], Wd[...], preferred_element_type=_f32) + bd[...]
    hu = jnp.dot(aggr, Ws[...], preferred_element_type=_f32) + bs[...]
    y = jnp.dot(jnp.concatenate([hv, hu], axis=1), Wu[...],
                preferred_element_type=_f32) + bu[...]
    m = jnp.mean(y, axis=0, keepdims=True)
    v = jnp.mean((y - m) ** 2, axis=0, keepdims=True)
    y = (y - m) / jnp.sqrt(v + 1.0) * g[...] + be[...]
    y = jnp.where(y >= 0, y, 0.01 * y)
    if with_post:
        y = jnp.dot(y, Wp[...], preferred_element_type=_f32) + bp[...]
    out[...] = y


def _full(a):
    return pl.BlockSpec(a.shape, lambda i: (0,) * a.ndim)


def _half(half, w):
    return pl.BlockSpec((N, w), lambda i: (half, 0))


def _tc_stage(with_post, half, x, sm, cnt, *args):
    width = 16 if with_post else D
    in_specs = ([_full(x), _half(half, D), _half(half, D)]
                + [_full(a) for a in args])
    return pl.pallas_call(
        functools.partial(_tc_body, with_post),
        grid=(1,),
        out_shape=jax.ShapeDtypeStruct((N, width), _f32),
        in_specs=in_specs,
        out_specs=pl.BlockSpec((N, width), lambda i: (0, 0)),
    )(x, sm, cnt, *args)


def kernel(x_n0, x_n1, edge_index_a, edge_index_b, W1a_src, b1a_src, W1a_dst, b1a_dst, W1a_upd, b1a_upd, W1b_src, b1b_src, W1b_dst, b1b_dst, W1b_upd, b1b_upd, W2a_src, b2a_src, W2a_dst, b2a_dst, W2a_upd, b2a_upd, W2b_src, b2b_src, W2b_dst, b2b_dst, W2b_upd, b2b_upd, g_bn1_n0, be_bn1_n0, g_bn1_n1, be_bn1_n1, g_bn2_n0, be_bn2_n0, g_bn2_n1, be_bn2_n1, W_post_n0, b_post_n0, W_post_n1, b_post_n1):
    def padidx(v, fill):
        v = v.astype(jnp.int32).reshape(NTILES, EPT)
        return jnp.pad(v, ((0, 0), (0, EPT_PAD - EPT)), constant_values=fill)

    si = jnp.stack([padidx(edge_index_a[0], 0),
                    padidx(edge_index_b[0] + N, 0)]
                   ).reshape(2, NTILES, NCHUNK, CHUNK)
    di = jnp.stack([padidx(edge_index_a[1], N),
                    padidx(edge_index_b[1], N)]
                   ).reshape(2, NTILES, NCHUNK, CHUNK)
    zf = jnp.zeros((N, D), _f32)
    row = lambda v: v.reshape(1, -1)

    xt1 = jnp.concatenate([x_n0, x_n1], axis=0)
    (sums1,) = _make_sc()(xt1, si, di, zf)
    (cnts,) = _make_cnt()(di, zf, jnp.ones((CHUNK, D), _f32))

    h1 = _tc_stage(False, 0, x_n1, sums1, cnts,
                   W1a_src, row(b1a_src), W1a_dst, row(b1a_dst),
                   W1a_upd, row(b1a_upd), row(g_bn1_n1), row(be_bn1_n1))
    h0 = _tc_stage(False, 1, x_n0, sums1, cnts,
                   W1b_src, row(b1b_src), W1b_dst, row(b1b_dst),
                   W1b_upd, row(b1b_upd), row(g_bn1_n0), row(be_bn1_n0))

    xt2 = jnp.concatenate([h0, h1], axis=0)
    (sums2,) = _make_sc()(xt2, si, di, zf)

    out1 = _tc_stage(True, 0, h1, sums2, cnts,
                     W2a_src, row(b2a_src), W2a_dst, row(b2a_dst),
                     W2a_upd, row(b2a_upd), row(g_bn2_n1), row(be_bn2_n1),
                     W_post_n1, row(b_post_n1))
    out0 = _tc_stage(True, 1, h0, sums2, cnts,
                     W2b_src, row(b2b_src), W2b_dst, row(b2b_dst),
                     W2b_upd, row(b2b_upd), row(g_bn2_n0), row(be_bn2_n0),
                     W_post_n0, row(b_post_n0))
    return out0, out1
